# baseline (speedup 1.0000x reference)
import functools

import jax
import jax.numpy as jnp
from jax import lax
from jax.experimental import pallas as pl
from jax.experimental.pallas import tpu as pltpu
from jax.experimental.pallas import tpu_sc as plsc

NC = 2
NS = 16
NW = NC * NS
LANES = 16

N_NODES = 50000
B = 1024
DEG = 32
D = 128
YEARS = 5
RELS = 2
PAIRS = YEARS * RELS
SEG_PER_W = B // NW
SEGS = PAIRS * SEG_PER_W
SEG_PAD = 384
CHUNK = 128
N_CHUNKS = SEG_PAD // CHUNK


def _sc_gather_sums(table, neigh):
    mesh = plsc.VectorSubcoreMesh(core_axis_name="c", subcore_axis_name="s")
    ids_per_pair = SEG_PER_W * DEG
    ids_per_w = PAIRS * ids_per_pair

    @functools.partial(
        pl.kernel,
        out_type=jax.ShapeDtypeStruct((PAIRS * B, D), jnp.float32),
        mesh=mesh,
        compiler_params=pltpu.CompilerParams(needs_layout_passes=False),
        scratch_types=[
            pltpu.VMEM((ids_per_w,), jnp.int32),
            pltpu.VMEM((DEG * N_CHUNKS, CHUNK), jnp.int32),
            pltpu.VMEM((SEG_PAD, D), jnp.float32),
            pltpu.SemaphoreType.DMA,
            pltpu.SemaphoreType.DMA,
        ],
    )
    def k(table_hbm, neigh_hbm, out_hbm, idx_v, idxt_v, acc_v, gsem, osem):
        wid = lax.axis_index("s") * NC + lax.axis_index("c")

        def stage(p, _):
            pltpu.async_copy(
                neigh_hbm.at[pl.ds(p * B * DEG + wid * ids_per_pair,
                                   ids_per_pair)],
                idx_v.at[pl.ds(p * ids_per_pair, ids_per_pair)], osem)
            return 0

        lax.fori_loop(0, PAIRS, stage, 0)

        def sdrain(p, _):
            pltpu.make_async_copy(neigh_hbm.at[pl.ds(0, ids_per_pair)],
                                  idx_v.at[pl.ds(0, ids_per_pair)],
                                  osem).wait()
            return 0

        lax.fori_loop(0, PAIRS, sdrain, 0)

        lanes = lax.iota(jnp.int32, LANES)

        def t_body(d, _):
            for c in range(N_CHUNKS):
                for g16 in range(CHUNK // LANES):
                    gg = lanes + (c * CHUNK + g16 * LANES)
                    valid = gg < SEGS
                    pos = (gg >> 5) * ids_per_pair + ((gg & 31) * DEG) + d
                    pos = jnp.where(valid, pos, 0)
                    off = (gg >> 6) * N_NODES
                    val = plsc.load_gather(idx_v, [pos])
                    val = jnp.where(valid, val + off, 0)
                    idxt_v[d * N_CHUNKS + c, pl.ds(g16 * LANES, LANES)] = val
            return 0

        lax.fori_loop(0, DEG, t_body, 0)

        z16 = jnp.zeros((LANES,), jnp.float32)

        def z_body(r, _):
            for v in range(D // LANES):
                acc_v[r, pl.ds(v * LANES, LANES)] = z16
            return 0

        lax.fori_loop(0, SEG_PAD, z_body, 0)

        def fire(t, _):
            c = t % N_CHUNKS
            pltpu.async_copy(table_hbm.at[idxt_v.at[t]],
                             acc_v.at[pl.ds(c * CHUNK, CHUNK)], gsem, add=True)
            return 0

        lax.fori_loop(0, DEG * N_CHUNKS, fire, 0)

        def drain(t, _):
            pltpu.make_async_copy(table_hbm.at[pl.ds(0, CHUNK)],
                                  acc_v.at[pl.ds(0, CHUNK)], gsem).wait()
            return 0

        lax.fori_loop(0, DEG * N_CHUNKS, drain, 0)

        def wr(p, _):
            pltpu.async_copy(
                acc_v.at[pl.ds(p * SEG_PER_W, SEG_PER_W)],
                out_hbm.at[pl.ds(p * B + wid * SEG_PER_W, SEG_PER_W)], osem)
            return 0

        lax.fori_loop(0, PAIRS, wr, 0)

        def wdrain(p, _):
            pltpu.make_async_copy(table_hbm.at[pl.ds(0, SEG_PER_W)],
                                  acc_v.at[pl.ds(0, SEG_PER_W)], osem).wait()
            return 0

        lax.fori_loop(0, PAIRS, wdrain, 0)

    return k(table, neigh)


def _tc_project(sums3, weights, weights_cite):
    def body(a_ref, b_ref, w_ref, wc_ref, o_ref):
        x0 = a_ref[0]
        x1 = b_ref[0]
        inv = jnp.float32(1.0 / DEG)
        w0 = (wc_ref[0] + wc_ref[1] + wc_ref[2]) * inv
        w1 = w_ref[1] * inv
        o_ref[...] = (jnp.dot(x0, w0, preferred_element_type=jnp.float32)
                      + jnp.dot(x1, w1, preferred_element_type=jnp.float32))[None]

    return pl.pallas_call(
        body,
        grid=(YEARS,),
        in_specs=[
            pl.BlockSpec((1, B, D), lambda i: (2 * i, 0, 0)),
            pl.BlockSpec((1, B, D), lambda i: (2 * i + 1, 0, 0)),
            pl.BlockSpec((RELS, D, D), lambda i: (0, 0, 0)),
            pl.BlockSpec((3, D, D), lambda i: (0, 0, 0)),
        ],
        out_specs=pl.BlockSpec((1, B, D), lambda i: (i, 0, 0)),
        out_shape=jax.ShapeDtypeStruct((YEARS, B, D), jnp.float32),
    )(sums3, sums3, weights, weights_cite)


def kernel(embeddings, train_year, neighbors, input_ids, weights, weights_cite):
    del train_year, input_ids
    years = embeddings.shape[0]
    table = embeddings.reshape(years * N_NODES, D)
    neigh = neighbors.reshape(PAIRS * B * DEG)
    sums = _sc_gather_sums(table, neigh)
    sums3 = sums.reshape(PAIRS, B, D)
    stacked = _tc_project(sums3, weights, weights_cite)
    return stacked.reshape(-1, years, D)


# speedup vs baseline: 21.7185x; 21.7185x over previous
import functools

import jax
import jax.numpy as jnp
from jax import lax
from jax.experimental import pallas as pl
from jax.experimental.pallas import tpu as pltpu
from jax.experimental.pallas import tpu_sc as plsc

NC = 2
NS = 16
NW = NC * NS
LANES = 16

N_NODES = 50000
B = 1024
DEG = 32
D = 128
YEARS = 5
RELS = 2
PAIRS = YEARS * RELS
SEG_PER_W = B // NW
ROWS_PER_CHUNK = 128
SEG_PER_CHUNK = ROWS_PER_CHUNK // DEG
CHUNKS_PER_PAIR = SEG_PER_W // SEG_PER_CHUNK
TOTAL_CHUNKS = PAIRS * CHUNKS_PER_PAIR
ACC_ROWS = PAIRS * SEG_PER_W
IDX_ROWS_PER_PAIR = B * DEG // ROWS_PER_CHUNK // NW


def _sc_gather_sums(table, neigh):
    mesh = plsc.VectorSubcoreMesh(core_axis_name="c", subcore_axis_name="s")

    @functools.partial(
        pl.kernel,
        out_type=jax.ShapeDtypeStruct((NW * ACC_ROWS, D), jnp.float32),
        mesh=mesh,
        scratch_types=[
            pltpu.VMEM((PAIRS, IDX_ROWS_PER_PAIR, D), jnp.int32),
            pltpu.VMEM((ROWS_PER_CHUNK, D), jnp.float32),
            pltpu.VMEM((ROWS_PER_CHUNK, D), jnp.float32),
            pltpu.VMEM((ROWS_PER_CHUNK, D), jnp.float32),
            pltpu.VMEM((ROWS_PER_CHUNK, D), jnp.float32),
            pltpu.VMEM((ACC_ROWS, D), jnp.float32),
            pltpu.SemaphoreType.DMA,
            pltpu.SemaphoreType.DMA,
            pltpu.SemaphoreType.DMA,
            pltpu.SemaphoreType.DMA,
            pltpu.SemaphoreType.DMA,
        ],
    )
    def k(table_hbm, neigh_hbm, out_hbm, idx_v, gb0, gb1, gb2, gb3, acc_v,
          sem0, sem1, sem2, sem3, osem):
        wid = lax.axis_index("s") * NC + lax.axis_index("c")

        pltpu.sync_copy(neigh_hbm.at[:, pl.ds(wid * IDX_ROWS_PER_PAIR,
                                              IDX_ROWS_PER_PAIR), :], idx_v)

        def off_body(q, _):
            off = (q // (2 * IDX_ROWS_PER_PAIR)) * N_NODES
            p = q // IDX_ROWS_PER_PAIR
            r = q % IDX_ROWS_PER_PAIR
            for v in range(D // LANES):
                sl = pl.ds(v * LANES, LANES)
                idx_v[p, r, sl] = idx_v[p, r, sl] + off
            return 0
        lax.fori_loop(0, PAIRS * IDX_ROWS_PER_PAIR, off_body, 0)

        def start(t, gb, sem):
            p = t // CHUNKS_PER_PAIR
            c = t % CHUNKS_PER_PAIR
            return pltpu.async_copy(table_hbm.at[idx_v.at[p, c]], gb, sem)

        def drain(gb, sem):
            pltpu.make_async_copy(table_hbm.at[pl.ds(0, ROWS_PER_CHUNK)],
                                  gb, sem).wait()

        zeros8 = tuple(jnp.zeros((LANES,), jnp.float32) for _ in range(D // LANES))

        def accum(gb, t):
            for s in range(SEG_PER_CHUNK):
                def d_body(dd, accs):
                    row = s * DEG + dd
                    return tuple(accs[v] + gb[row, pl.ds(v * LANES, LANES)]
                                 for v in range(D // LANES))
                accs = lax.fori_loop(0, DEG, d_body, zeros8, unroll=8)
                for v in range(D // LANES):
                    acc_v[t * SEG_PER_CHUNK + s, pl.ds(v * LANES, LANES)] = accs[v]

        bufs = ((gb0, sem0), (gb1, sem1), (gb2, sem2), (gb3, sem3))
        for kb, (gb, sem) in enumerate(bufs):
            start(kb, gb, sem)

        def pipe(g, _):
            t0 = 4 * g
            for kb, (gb, sem) in enumerate(bufs):
                t = t0 + kb
                drain(gb, sem)
                accum(gb, t)

                @pl.when(t + 4 < TOTAL_CHUNKS)
                def _():
                    start(t + 4, gb, sem)
            return 0

        lax.fori_loop(0, TOTAL_CHUNKS // 4, pipe, 0)

        pltpu.async_copy(acc_v, out_hbm.at[pl.ds(wid * ACC_ROWS, ACC_ROWS)],
                         osem).wait()

    return k(table, neigh)


def _tc_project(sums4, weights, weights_cite):
    def body(a_ref, b_ref, w_ref, wc_ref, o_ref):
        x0 = a_ref[...].reshape(B, D)
        x1 = b_ref[...].reshape(B, D)
        inv = jnp.float32(1.0 / DEG)
        w0 = (wc_ref[0] + wc_ref[1] + wc_ref[2]) * inv
        w1 = w_ref[1] * inv
        o_ref[...] = (jnp.dot(x0, w0, preferred_element_type=jnp.float32)
                      + jnp.dot(x1, w1, preferred_element_type=jnp.float32))[None]

    return pl.pallas_call(
        body,
        grid=(YEARS,),
        in_specs=[
            pl.BlockSpec((NW, 1, SEG_PER_W, D), lambda i: (0, 2 * i, 0, 0)),
            pl.BlockSpec((NW, 1, SEG_PER_W, D), lambda i: (0, 2 * i + 1, 0, 0)),
            pl.BlockSpec((RELS, D, D), lambda i: (0, 0, 0)),
            pl.BlockSpec((3, D, D), lambda i: (0, 0, 0)),
        ],
        out_specs=pl.BlockSpec((1, B, D), lambda i: (i, 0, 0)),
        out_shape=jax.ShapeDtypeStruct((YEARS, B, D), jnp.float32),
    )(sums4, sums4, weights, weights_cite)


def kernel(embeddings, train_year, neighbors, input_ids, weights, weights_cite):
    del train_year, input_ids
    years = embeddings.shape[0]
    table = embeddings.reshape(years * N_NODES, D)
    neigh = neighbors.reshape(PAIRS, B * DEG // D, D)
    sums = _sc_gather_sums(table, neigh)
    sums4 = sums.reshape(NW, PAIRS, SEG_PER_W, D)
    stacked = _tc_project(sums4, weights, weights_cite)
    return stacked.reshape(-1, years, D)
